# baseline (speedup 1.0000x reference)
import jax
import jax.numpy as jnp
from jax.experimental import pallas as pl
from jax.experimental.pallas import tpu as pltpu

_BN_EPS = 1e-5


def _make_stats_kernel(c_in, nb):
    def _stats_kernel(x_ref, p_ref):
        sums = {}
        ssum = [None] * c_in
        for b in range(nb):
            xs = [x_ref[b, i] for i in range(c_in)]
            for i in range(c_in):
                for j in range(i, c_in):
                    p = jnp.sum(xs[i] * xs[j], axis=0)
                    sums[(i, j)] = p if b == 0 else sums[(i, j)] + p
                q = jnp.sum(xs[i], axis=0)
                ssum[i] = q if b == 0 else ssum[i] + q
        rows = [sums[(min(i, j), max(i, j))]
                for i in range(c_in) for j in range(c_in)]
        p_ref[0] = jnp.stack(rows + ssum)
    return _stats_kernel


def _make_apply_kernel(c_in, c_out):
    def _apply_kernel(x_ref, w_ref, shift_ref, o_ref):
        xs = [x_ref[0, i] for i in range(c_in)]
        for o in range(c_out):
            acc = xs[0] * w_ref[o, 0] + shift_ref[o]
            for i in range(1, c_in):
                acc += xs[i] * w_ref[o, i]
            o_ref[0, o] = acc
    return _apply_kernel


def kernel(x_nchw, conv_weight, gamma, beta):
    n, c_in, h, w_sp = x_nchw.shape
    c_out = conv_weight.shape[0]
    m = n * h * w_sp
    itemsize = jnp.dtype(x_nchw.dtype).itemsize
    w_mat = conv_weight[:, :, 0, 0].astype(jnp.float32)
    nrows = c_in * c_in + c_in

    nb = 16 if n % 16 == 0 else (4 if n % 4 == 0 else 1)
    g1 = n // nb
    partials = pl.pallas_call(
        _make_stats_kernel(c_in, nb),
        out_shape=jax.ShapeDtypeStruct((g1, nrows, w_sp), jnp.float32),
        grid=(g1,),
        in_specs=[
            pl.BlockSpec((nb, c_in, h, w_sp), lambda i: (i, 0, 0, 0)),
        ],
        out_specs=pl.BlockSpec((1, nrows, w_sp), lambda i: (i, 0, 0)),
        compiler_params=pltpu.CompilerParams(
            dimension_semantics=("parallel",)),
        cost_estimate=pl.CostEstimate(
            flops=2 * m * (c_in * (c_in + 1) // 2 + c_in),
            transcendentals=0,
            bytes_accessed=itemsize * m * c_in + 4 * g1 * nrows * w_sp),
    )(x_nchw)

    red = jnp.sum(partials, axis=(0, 2))
    g = red[:c_in * c_in].reshape(c_in, c_in)
    s = red[c_in * c_in:]
    mean = (w_mat @ s) / m
    ey2 = jnp.sum((w_mat @ g) * w_mat, axis=1) / m
    var = jnp.maximum(ey2 - mean * mean, 0.0)
    inv_std = jax.lax.rsqrt(var + _BN_EPS)
    scale = gamma.astype(jnp.float32) * inv_std
    shift = beta.astype(jnp.float32) - mean * scale
    w_scaled = w_mat * scale[:, None]

    hb = 256 if h % 256 == 0 else (128 if h % 128 == 0 else h)
    num_t = h // hb
    out = pl.pallas_call(
        _make_apply_kernel(c_in, c_out),
        out_shape=jax.ShapeDtypeStruct((n, c_out, h, w_sp), x_nchw.dtype),
        grid=(n, num_t),
        in_specs=[
            pl.BlockSpec((1, c_in, hb, w_sp), lambda b, t: (b, 0, t, 0)),
            pl.BlockSpec(memory_space=pltpu.SMEM),
            pl.BlockSpec(memory_space=pltpu.SMEM),
        ],
        out_specs=pl.BlockSpec((1, c_out, hb, w_sp),
                               lambda b, t: (b, 0, t, 0)),
        compiler_params=pltpu.CompilerParams(
            dimension_semantics=("parallel", "parallel")),
        cost_estimate=pl.CostEstimate(
            flops=2 * m * c_in * c_out + m * c_out,
            transcendentals=0,
            bytes_accessed=itemsize * (m * c_in + m * c_out)
            + 4 * (c_in + 1) * c_out),
    )(x_nchw, w_scaled, shift)

    return out


# speedup vs baseline: 1.0374x; 1.0374x over previous
import jax
import jax.numpy as jnp
from jax.experimental import pallas as pl
from jax.experimental.pallas import tpu as pltpu

_BN_EPS = 1e-5


def _make_stats_kernel(c_in, nb):
    def _stats_kernel(x_ref, p_ref):
        sums = {}
        ssum = [None] * c_in
        for b in range(nb):
            xs = [x_ref[b, i] for i in range(c_in)]
            for i in range(c_in):
                for j in range(i, c_in):
                    p = jnp.sum(xs[i] * xs[j], axis=0)
                    sums[(i, j)] = p if b == 0 else sums[(i, j)] + p
                q = jnp.sum(xs[i], axis=0)
                ssum[i] = q if b == 0 else ssum[i] + q
        rows = [sums[(min(i, j), max(i, j))]
                for i in range(c_in) for j in range(c_in)]
        p_ref[0] = jnp.stack(rows + ssum)
    return _stats_kernel


def _make_apply_kernel(c_in, c_out, ba):
    def _apply_kernel(x_ref, w_ref, shift_ref, o_ref):
        for b in range(ba):
            xs = [x_ref[b, i] for i in range(c_in)]
            for o in range(c_out):
                acc = xs[0] * w_ref[o, 0] + shift_ref[o]
                for i in range(1, c_in):
                    acc += xs[i] * w_ref[o, i]
                o_ref[b, o] = acc
    return _apply_kernel


def kernel(x_nchw, conv_weight, gamma, beta):
    n, c_in, h, w_sp = x_nchw.shape
    c_out = conv_weight.shape[0]
    m = n * h * w_sp
    itemsize = jnp.dtype(x_nchw.dtype).itemsize
    w_mat = conv_weight[:, :, 0, 0].astype(jnp.float32)
    nrows = c_in * c_in + c_in

    nb = 8 if n % 8 == 0 else (4 if n % 4 == 0 else 1)
    g1 = n // nb
    partials = pl.pallas_call(
        _make_stats_kernel(c_in, nb),
        out_shape=jax.ShapeDtypeStruct((g1, nrows, w_sp), jnp.float32),
        grid=(g1,),
        in_specs=[
            pl.BlockSpec((nb, c_in, h, w_sp), lambda i: (i, 0, 0, 0)),
        ],
        out_specs=pl.BlockSpec((1, nrows, w_sp), lambda i: (i, 0, 0)),
        compiler_params=pltpu.CompilerParams(
            dimension_semantics=("parallel",)),
        cost_estimate=pl.CostEstimate(
            flops=2 * m * (c_in * (c_in + 1) // 2 + c_in),
            transcendentals=0,
            bytes_accessed=itemsize * m * c_in + 4 * g1 * nrows * w_sp),
    )(x_nchw)

    red = jnp.sum(partials, axis=(0, 2))
    g = red[:c_in * c_in].reshape(c_in, c_in)
    s = red[c_in * c_in:]
    mean = (w_mat @ s) / m
    ey2 = jnp.sum((w_mat @ g) * w_mat, axis=1) / m
    var = jnp.maximum(ey2 - mean * mean, 0.0)
    inv_std = jax.lax.rsqrt(var + _BN_EPS)
    scale = gamma.astype(jnp.float32) * inv_std
    shift = beta.astype(jnp.float32) - mean * scale
    w_scaled = w_mat * scale[:, None]

    hb = 256 if h % 256 == 0 else (128 if h % 128 == 0 else h)
    num_t = h // hb
    ba = 2 if (n % 2 == 0 and hb == h) else 1
    out = pl.pallas_call(
        _make_apply_kernel(c_in, c_out, ba),
        out_shape=jax.ShapeDtypeStruct((n, c_out, h, w_sp), x_nchw.dtype),
        grid=(n // ba, num_t),
        in_specs=[
            pl.BlockSpec((ba, c_in, hb, w_sp), lambda b, t: (b, 0, t, 0)),
            pl.BlockSpec(memory_space=pltpu.SMEM),
            pl.BlockSpec(memory_space=pltpu.SMEM),
        ],
        out_specs=pl.BlockSpec((ba, c_out, hb, w_sp),
                               lambda b, t: (b, 0, t, 0)),
        compiler_params=pltpu.CompilerParams(
            dimension_semantics=("parallel", "parallel")),
        cost_estimate=pl.CostEstimate(
            flops=2 * m * c_in * c_out + m * c_out,
            transcendentals=0,
            bytes_accessed=itemsize * (m * c_in + m * c_out)
            + 4 * (c_in + 1) * c_out),
    )(x_nchw, w_scaled, shift)

    return out
